# baseline (speedup 1.0000x reference)
import functools

import jax
import jax.numpy as jnp
from jax import lax
from jax.experimental import pallas as pl
from jax.experimental.pallas import tpu as pltpu
from jax.experimental.pallas import tpu_sc as plsc

_LOG2 = 0.6931471805599453
_NC = 2
_NS = 16
_NW = _NC * _NS


def _ssp(v):
    return jnp.maximum(v, 0.0) + jnp.log1p(jnp.exp(-jnp.abs(v))) - _LOG2


def _pack_bf16(a):
    hw = a.shape[1] // 2
    rn = jax.lax.bitcast_convert_type(
        a.astype(jnp.bfloat16).astype(jnp.float32), jnp.int32)
    lo = jax.lax.shift_right_logical(rn[:, :hw], jnp.int32(16))
    hi = rn[:, hw:] & jnp.int32(-65536)
    return lo | hi


def _unpack_sum(u):
    f32 = jnp.float32
    hw = u.shape[1] // 2
    lo = jax.lax.bitcast_convert_type(u << 16, f32)
    hi = jax.lax.bitcast_convert_type(u & jnp.int32(-65536), f32)
    return jnp.concatenate([lo[:, :hw] + lo[:, hw:], hi[:, :hw] + hi[:, hw:]],
                           axis=1)


def _prep_body(x_ref, we_ref, be_ref, wi_ref, bi_ref, wa_ref, wb_ref,
               h0_ref, h_ref, a_ref, b_ref):
    f32 = jnp.float32
    h0 = jnp.dot(x_ref[...], we_ref[...], preferred_element_type=f32) + be_ref[...]
    h = jnp.dot(h0, wi_ref[...], preferred_element_type=f32) + bi_ref[...]
    h0_ref[...] = h0
    h_ref[...] = h
    a = jnp.dot(h, wa_ref[...], preferred_element_type=f32)
    b = jnp.dot(h, wb_ref[...], preferred_element_type=f32)
    a_ref[...] = _pack_bf16(a)
    b_ref[...] = _pack_bf16(b)


def _prep(x, W_emb, b_emb, W_in, b_in, We1a, We1b, bn):
    n, d = x.shape
    h = W_emb.shape[1]
    grid = n // bn
    full = lambda s: pl.BlockSpec(s, lambda i: (0, 0))
    out_f = jax.ShapeDtypeStruct((n, h), jnp.float32)
    out_p = jax.ShapeDtypeStruct((n, h // 2), jnp.int32)
    return pl.pallas_call(
        _prep_body,
        grid=(grid,),
        in_specs=[
            pl.BlockSpec((bn, d), lambda i: (i, 0)),
            full((d, h)), full((1, h)), full((h, h)), full((1, h)),
            full((h, h)), full((h, h)),
        ],
        out_specs=[pl.BlockSpec((bn, h), lambda i: (i, 0)),
                   pl.BlockSpec((bn, h), lambda i: (i, 0)),
                   pl.BlockSpec((bn, h // 2), lambda i: (i, 0)),
                   pl.BlockSpec((bn, h // 2), lambda i: (i, 0))],
        out_shape=[out_f, out_f, out_p, out_p],
    )(x, W_emb, b_emb.reshape(1, h), W_in, b_in.reshape(1, h), We1a, We1b)


def _sc_gather(A, B, row, col, px, py, pz, sb):
    n, hw = A.shape
    e = row.shape[0]
    ch = e // _NW
    nblk = ch // sb
    btail = ch % sb != 0
    g16 = ch // 16
    gtail = ch % 16 != 0
    mesh = plsc.VectorSubcoreMesh(core_axis_name="c", subcore_axis_name="s")

    @functools.partial(
        pl.kernel,
        out_type=(jax.ShapeDtypeStruct((e, 2 * hw), jnp.int32),
                  jax.ShapeDtypeStruct((e,), jnp.float32)),
        mesh=mesh,
        scratch_types=[
            pltpu.VMEM((ch,), jnp.int32),
            pltpu.VMEM((ch,), jnp.int32),
            pltpu.VMEM((n,), jnp.float32),
            pltpu.VMEM((n,), jnp.float32),
            pltpu.VMEM((n,), jnp.float32),
            pltpu.VMEM((ch,), jnp.float32),
            pltpu.VMEM((sb, hw), jnp.int32),
            pltpu.VMEM((sb, hw), jnp.int32),
            pltpu.SemaphoreType.DMA,
            pltpu.SemaphoreType.DMA,
        ],
        compiler_params=pltpu.CompilerParams(needs_layout_passes=False,
                                             use_tc_tiling_on_sc=False),
    )
    def k(a_hbm, b_hbm, row_hbm, col_hbm, px_hbm, py_hbm, pz_hbm,
          ab_hbm, rad_hbm,
          row_v, col_v, px_v, py_v, pz_v, rad_v, buf_a, buf_b, sem_a, sem_b):
        cid = lax.axis_index("c")
        sid = lax.axis_index("s")
        wid = sid * _NC + cid
        base = wid * ch
        pltpu.sync_copy(row_hbm.at[pl.ds(base, ch)], row_v)
        pltpu.sync_copy(col_hbm.at[pl.ds(base, ch)], col_v)
        pltpu.sync_copy(px_hbm, px_v)
        pltpu.sync_copy(py_hbm, py_v)
        pltpu.sync_copy(pz_hbm, pz_v)

        def rad_at(off):
            ir = row_v[pl.ds(off, 16)]
            ic = col_v[pl.ds(off, 16)]
            dx = plsc.load_gather(px_v, [ir]) - plsc.load_gather(px_v, [ic])
            dy = plsc.load_gather(py_v, [ir]) - plsc.load_gather(py_v, [ic])
            dz = plsc.load_gather(pz_v, [ir]) - plsc.load_gather(pz_v, [ic])
            rad_v[pl.ds(off, 16)] = dx * dx + dy * dy + dz * dz

        def rad_body(i, carry):
            rad_at(i * 16)
            return carry

        lax.fori_loop(0, g16, rad_body, 0)
        if gtail:
            rad_at(ch - 16)
        pltpu.sync_copy(rad_v, rad_hbm.at[pl.ds(base, ch)])

        def blk_at(off):
            ca = pltpu.async_copy(a_hbm.at[row_v.at[pl.ds(off, sb)]], buf_a, sem_a)
            cb = pltpu.async_copy(b_hbm.at[col_v.at[pl.ds(off, sb)]], buf_b, sem_b)
            ca.wait()
            cb.wait()
            pltpu.sync_copy(buf_a, ab_hbm.at[pl.ds(base + off, sb), pl.ds(0, hw)])
            pltpu.sync_copy(buf_b, ab_hbm.at[pl.ds(base + off, sb), pl.ds(hw, hw)])

        def blk_body(bidx, carry):
            blk_at(bidx * sb)
            return carry

        lax.fori_loop(0, nblk, blk_body, 0)
        if btail:
            blk_at(ch - sb)

    return k(A, B, row, col, px, py, pz)


def _emlp_body(ab_ref, rad_ref, ea_ref, wr_ref, wat_ref, b1_ref,
               we2_ref, b2_ref, m2_ref):
    f32 = jnp.float32
    m1 = (_unpack_sum(ab_ref[...])
          + rad_ref[...] * wr_ref[...]
          + jnp.dot(ea_ref[...], wat_ref[...], preferred_element_type=f32)
          + b1_ref[...])
    em = _ssp(m1)
    m2_ref[...] = _ssp(jnp.dot(em, we2_ref[...], preferred_element_type=f32)
                       + b2_ref[...])


def _emlp(ab, rad, ea, w_r, W_at, b_e1, W_e2, b_e2, be):
    e, h = ab.shape
    enf = ea.shape[1]
    grid = e // be
    full = lambda s: pl.BlockSpec(s, lambda i: (0, 0))
    return pl.pallas_call(
        _emlp_body,
        grid=(grid,),
        in_specs=[
            pl.BlockSpec((be, h), lambda i: (i, 0)),
            pl.BlockSpec((be, 1), lambda i: (i, 0)),
            pl.BlockSpec((be, enf), lambda i: (i, 0)),
            full((1, h)), full((enf, h)), full((1, h)),
            full((h, h)), full((1, h)),
        ],
        out_specs=pl.BlockSpec((be, h), lambda i: (i, 0)),
        out_shape=jax.ShapeDtypeStruct((e, h), jnp.float32),
    )(ab, rad.reshape(e, 1), ea, w_r.reshape(1, h), W_at,
      b_e1.reshape(1, h), W_e2, b_e2.reshape(1, h))


def _sc_scatter(m2s, row3ds, n):
    nseg = len(m2s)
    e, h = m2s[0].shape
    ch = e // _NW
    nblk, sb = row3ds[0].shape[1], row3ds[0].shape[2]
    nb_n = n // sb
    nrounds = -(-nb_n // _NS)
    mesh = plsc.VectorSubcoreMesh(core_axis_name="c", subcore_axis_name="s")

    @functools.partial(
        pl.kernel,
        out_type=jax.ShapeDtypeStruct((_NC, n, h), jnp.float32),
        mesh=mesh,
        scratch_types=[
            pltpu.VMEM((nblk, sb), jnp.int32),
            pltpu.VMEM((sb, h), jnp.float32),
            pltpu.VMEM((sb, h), jnp.float32),
            pltpu.VMEM_SHARED((n, h), jnp.float32),
        ],
    )
    def k(*refs):
        m2_hbms = refs[:nseg]
        row3d_hbms = refs[nseg:2 * nseg]
        agg_hbm = refs[2 * nseg]
        idx_v, buf_m, zbuf, agg_sh = refs[2 * nseg + 1:]
        cid = lax.axis_index("c")
        sid = lax.axis_index("s")
        wid = sid * _NC + cid
        base = wid * ch

        zeros16 = jnp.zeros((16,), jnp.float32)

        def z_body(i, carry):
            for j in range(h // 16):
                zbuf[i, pl.ds(j * 16, 16)] = zeros16
            return carry

        lax.fori_loop(0, sb, z_body, 0)
        for t in range(nrounds):
            blk = sid + t * _NS
            if nb_n % _NS == 0:
                pltpu.sync_copy(zbuf, agg_sh.at[pl.ds(blk * sb, sb)])
            else:
                @pl.when(blk < nb_n)
                def _():
                    pltpu.sync_copy(zbuf, agg_sh.at[pl.ds(blk * sb, sb)])
        plsc.subcore_barrier()

        for m2_hbm, row3d_hbm in zip(m2_hbms, row3d_hbms):
            pltpu.sync_copy(row3d_hbm.at[wid], idx_v)

            def s_body(j, carry):
                pltpu.sync_copy(m2_hbm.at[pl.ds(base + j * sb, sb)], buf_m)
                pltpu.sync_copy(buf_m, agg_sh.at[idx_v.at[j]], add=True)
                return carry

            lax.fori_loop(0, nblk, s_body, 0)
        plsc.subcore_barrier()
        for t in range(nrounds):
            blk = sid + t * _NS
            if nb_n % _NS == 0:
                pltpu.sync_copy(agg_sh.at[pl.ds(blk * sb, sb)],
                                agg_hbm.at[cid, pl.ds(blk * sb, sb)])
            else:
                @pl.when(blk < nb_n)
                def _():
                    pltpu.sync_copy(agg_sh.at[pl.ds(blk * sb, sb)],
                                    agg_hbm.at[cid, pl.ds(blk * sb, sb)])

    return k(*m2s, *row3ds)


def _node_body(h0_ref, h_ref, *rest):
    f32 = jnp.float32
    agg_refs = rest[:-8]
    (wnh_ref, wna_ref, b1_ref, wn2_ref, b2_ref, wo_ref, bo_ref,
     out_ref) = rest[-8:]
    agg = agg_refs[0][0] + agg_refs[0][1]
    for r in agg_refs[1:]:
        agg = agg + r[0] + r[1]
    t = _ssp(jnp.dot(h_ref[...], wnh_ref[...], preferred_element_type=f32)
             + jnp.dot(agg, wna_ref[...], preferred_element_type=f32)
             + b1_ref[...])
    hn = jnp.dot(t, wn2_ref[...], preferred_element_type=f32) + b2_ref[...]
    hn = jnp.dot(hn, wo_ref[...], preferred_element_type=f32) + bo_ref[...]
    h0 = h0_ref[...]
    out_ref[0] = h0
    out_ref[1] = h0 + hn


def _node(h0, hfeat, aggps, Wn1h, Wn1a, b_n1, W_n2, b_n2, W_out, b_out, bn):
    n, h = h0.shape
    grid = n // bn
    full = lambda s: pl.BlockSpec(s, lambda i: (0, 0))
    return pl.pallas_call(
        _node_body,
        grid=(grid,),
        in_specs=[
            pl.BlockSpec((bn, h), lambda i: (i, 0)),
            pl.BlockSpec((bn, h), lambda i: (i, 0)),
        ] + [pl.BlockSpec((_NC, bn, h), lambda i: (0, i, 0))] * len(aggps) + [
            full((h, h)), full((h, h)), full((1, h)),
            full((h, h)), full((1, h)), full((h, h)), full((1, h)),
        ],
        out_specs=pl.BlockSpec((2, bn, h), lambda i: (0, i, 0)),
        out_shape=jax.ShapeDtypeStruct((2, n, h), jnp.float32),
    )(h0, hfeat, *aggps, Wn1h, Wn1a, b_n1.reshape(1, h), W_n2,
      b_n2.reshape(1, h), W_out, b_out.reshape(1, h))


def kernel(x, edge_index, pos, edge_attr,
           W_emb, b_emb, W_in, b_in,
           W_e1, b_e1, W_e2, b_e2,
           W_n1, b_n1, W_n2, b_n2,
           W_c1, b_c1, W_c2,
           W_out, b_out):
    n, d = x.shape
    h = W_emb.shape[1]
    e = edge_index.shape[1]
    nseg = 2
    eseg = e // nseg
    sbg = 80
    sbs = 40

    row = edge_index[0].astype(jnp.int32)
    col = edge_index[1].astype(jnp.int32)
    px = pos[:, 0].astype(jnp.float32)
    py = pos[:, 1].astype(jnp.float32)
    pz = pos[:, 2].astype(jnp.float32)

    We1a = W_e1[:h]
    We1b = W_e1[h:2 * h]
    w_r = W_e1[2 * h]
    W_at = W_e1[2 * h + 1:]

    h0, hfeat, A, B = _prep(x, W_emb, b_emb, W_in, b_in, We1a, We1b, bn=1000)
    m2s, row3ds = [], []
    for s in range(nseg):
        sl = slice(s * eseg, (s + 1) * eseg)
        row_s, col_s = row[sl], col[sl]
        ab, rad = _sc_gather(A, B, row_s, col_s, px, py, pz, sbg)
        m2s.append(_emlp(ab, rad, edge_attr[sl], w_r, W_at, b_e1, W_e2, b_e2,
                         be=2000))
        row3ds.append(row_s.reshape(_NW, eseg // (_NW * sbs), sbs))
    aggp = _sc_scatter(m2s, row3ds, n)
    return _node(h0, hfeat, [aggp], W_n1[:h], W_n1[h:], b_n1, W_n2, b_n2,
                 W_out, b_out, bn=1000)


# speedup vs baseline: 1.2199x; 1.2199x over previous
import functools

import jax
import jax.numpy as jnp
from jax import lax
from jax.experimental import pallas as pl
from jax.experimental.pallas import tpu as pltpu
from jax.experimental.pallas import tpu_sc as plsc

_LOG2 = 0.6931471805599453
_NC = 2
_NS = 16
_NW = _NC * _NS


def _ssp(v):
    return jnp.maximum(v, 0.0) + jnp.log1p(jnp.exp(-jnp.abs(v))) - _LOG2


def _pack_bf16(a):
    hw = a.shape[1] // 2
    rn = jax.lax.bitcast_convert_type(
        a.astype(jnp.bfloat16).astype(jnp.float32), jnp.int32)
    lo = jax.lax.shift_right_logical(rn[:, :hw], jnp.int32(16))
    hi = rn[:, hw:] & jnp.int32(-65536)
    return lo | hi


def _unpack_sum(u):
    f32 = jnp.float32
    hw = u.shape[1] // 2
    lo = jax.lax.bitcast_convert_type(u << 16, f32)
    hi = jax.lax.bitcast_convert_type(u & jnp.int32(-65536), f32)
    return jnp.concatenate([lo[:, :hw] + lo[:, hw:], hi[:, :hw] + hi[:, hw:]],
                           axis=1)


def _prep_body(x_ref, we_ref, be_ref, wi_ref, bi_ref, wa_ref, wb_ref,
               h0_ref, h_ref, a_ref, b_ref):
    f32 = jnp.float32
    h0 = jnp.dot(x_ref[...], we_ref[...], preferred_element_type=f32) + be_ref[...]
    h = jnp.dot(h0, wi_ref[...], preferred_element_type=f32) + bi_ref[...]
    h0_ref[...] = h0
    h_ref[...] = h
    a = jnp.dot(h, wa_ref[...], preferred_element_type=f32)
    b = jnp.dot(h, wb_ref[...], preferred_element_type=f32)
    a_ref[...] = _pack_bf16(a)
    b_ref[...] = _pack_bf16(b)


def _prep(x, W_emb, b_emb, W_in, b_in, We1a, We1b, bn):
    n, d = x.shape
    h = W_emb.shape[1]
    grid = n // bn
    full = lambda s: pl.BlockSpec(s, lambda i: (0, 0))
    out_f = jax.ShapeDtypeStruct((n, h), jnp.float32)
    out_p = jax.ShapeDtypeStruct((n, h // 2), jnp.int32)
    return pl.pallas_call(
        _prep_body,
        grid=(grid,),
        in_specs=[
            pl.BlockSpec((bn, d), lambda i: (i, 0)),
            full((d, h)), full((1, h)), full((h, h)), full((1, h)),
            full((h, h)), full((h, h)),
        ],
        out_specs=[pl.BlockSpec((bn, h), lambda i: (i, 0)),
                   pl.BlockSpec((bn, h), lambda i: (i, 0)),
                   pl.BlockSpec((bn, h // 2), lambda i: (i, 0)),
                   pl.BlockSpec((bn, h // 2), lambda i: (i, 0))],
        out_shape=[out_f, out_f, out_p, out_p],
    )(x, W_emb, b_emb.reshape(1, h), W_in, b_in.reshape(1, h), We1a, We1b)


def _sc_gather(A, B, row, col, px, py, pz, sb):
    n, hw = A.shape
    e = row.shape[0]
    ch = e // _NW
    nblk = ch // sb
    btail = ch % sb != 0
    g16 = ch // 16
    gtail = ch % 16 != 0
    mesh = plsc.VectorSubcoreMesh(core_axis_name="c", subcore_axis_name="s")

    @functools.partial(
        pl.kernel,
        out_type=(jax.ShapeDtypeStruct((e, 2 * hw), jnp.int32),
                  jax.ShapeDtypeStruct((e,), jnp.float32)),
        mesh=mesh,
        scratch_types=[
            pltpu.VMEM((ch,), jnp.int32),
            pltpu.VMEM((ch,), jnp.int32),
            pltpu.VMEM((n,), jnp.float32),
            pltpu.VMEM((n,), jnp.float32),
            pltpu.VMEM((n,), jnp.float32),
            pltpu.VMEM((ch,), jnp.float32),
            pltpu.VMEM((sb, hw), jnp.int32),
            pltpu.VMEM((sb, hw), jnp.int32),
            pltpu.VMEM((sb, hw), jnp.int32),
            pltpu.VMEM((sb, hw), jnp.int32),
            pltpu.SemaphoreType.DMA,
            pltpu.SemaphoreType.DMA,
            pltpu.SemaphoreType.DMA,
            pltpu.SemaphoreType.DMA,
        ],
        compiler_params=pltpu.CompilerParams(needs_layout_passes=False,
                                             use_tc_tiling_on_sc=False),
    )
    def k(a_hbm, b_hbm, row_hbm, col_hbm, px_hbm, py_hbm, pz_hbm,
          ab_hbm, rad_hbm,
          row_v, col_v, px_v, py_v, pz_v, rad_v,
          buf_a0, buf_b0, buf_a1, buf_b1, sem_a0, sem_b0, sem_a1, sem_b1):
        cid = lax.axis_index("c")
        sid = lax.axis_index("s")
        wid = sid * _NC + cid
        base = wid * ch
        pltpu.sync_copy(row_hbm.at[pl.ds(base, ch)], row_v)
        pltpu.sync_copy(col_hbm.at[pl.ds(base, ch)], col_v)
        pltpu.sync_copy(px_hbm, px_v)
        pltpu.sync_copy(py_hbm, py_v)
        pltpu.sync_copy(pz_hbm, pz_v)

        def rad_at(off):
            ir = row_v[pl.ds(off, 16)]
            ic = col_v[pl.ds(off, 16)]
            dx = plsc.load_gather(px_v, [ir]) - plsc.load_gather(px_v, [ic])
            dy = plsc.load_gather(py_v, [ir]) - plsc.load_gather(py_v, [ic])
            dz = plsc.load_gather(pz_v, [ir]) - plsc.load_gather(pz_v, [ic])
            rad_v[pl.ds(off, 16)] = dx * dx + dy * dy + dz * dz

        def rad_body(i, carry):
            rad_at(i * 16)
            return carry

        lax.fori_loop(0, g16, rad_body, 0)
        if gtail:
            rad_at(ch - 16)
        pltpu.sync_copy(rad_v, rad_hbm.at[pl.ds(base, ch)])

        def issue(off, buf_a, buf_b, sem_a, sem_b):
            ca = pltpu.async_copy(a_hbm.at[row_v.at[pl.ds(off, sb)]], buf_a, sem_a)
            cb = pltpu.async_copy(b_hbm.at[col_v.at[pl.ds(off, sb)]], buf_b, sem_b)
            return ca, cb

        def drain(off, ca, cb, buf_a, buf_b):
            ca.wait()
            cb.wait()
            pltpu.sync_copy(buf_a, ab_hbm.at[pl.ds(base + off, sb), pl.ds(0, hw)])
            pltpu.sync_copy(buf_b, ab_hbm.at[pl.ds(base + off, sb), pl.ds(hw, hw)])

        npair = nblk // 2

        def blk_body(i, carry):
            o0 = (2 * i) * sb
            o1 = (2 * i + 1) * sb
            c0 = issue(o0, buf_a0, buf_b0, sem_a0, sem_b0)
            c1 = issue(o1, buf_a1, buf_b1, sem_a1, sem_b1)
            drain(o0, *c0, buf_a0, buf_b0)
            drain(o1, *c1, buf_a1, buf_b1)
            return carry

        lax.fori_loop(0, npair, blk_body, 0)
        for j in range(2 * npair, nblk):
            c0 = issue(j * sb, buf_a0, buf_b0, sem_a0, sem_b0)
            drain(j * sb, *c0, buf_a0, buf_b0)
        if btail:
            c0 = issue(ch - sb, buf_a0, buf_b0, sem_a0, sem_b0)
            drain(ch - sb, *c0, buf_a0, buf_b0)

    return k(A, B, row, col, px, py, pz)


def _emlp_body(ab_ref, rad_ref, ea_ref, wr_ref, wat_ref, b1_ref,
               we2_ref, b2_ref, m2_ref):
    f32 = jnp.float32
    m1 = (_unpack_sum(ab_ref[...])
          + rad_ref[...] * wr_ref[...]
          + jnp.dot(ea_ref[...], wat_ref[...], preferred_element_type=f32)
          + b1_ref[...])
    em = _ssp(m1)
    m2_ref[...] = _ssp(jnp.dot(em, we2_ref[...], preferred_element_type=f32)
                       + b2_ref[...])


def _emlp(ab, rad, ea, w_r, W_at, b_e1, W_e2, b_e2, be):
    e, h = ab.shape
    enf = ea.shape[1]
    grid = e // be
    full = lambda s: pl.BlockSpec(s, lambda i: (0, 0))
    return pl.pallas_call(
        _emlp_body,
        grid=(grid,),
        in_specs=[
            pl.BlockSpec((be, h), lambda i: (i, 0)),
            pl.BlockSpec((be, 1), lambda i: (i, 0)),
            pl.BlockSpec((be, enf), lambda i: (i, 0)),
            full((1, h)), full((enf, h)), full((1, h)),
            full((h, h)), full((1, h)),
        ],
        out_specs=pl.BlockSpec((be, h), lambda i: (i, 0)),
        out_shape=jax.ShapeDtypeStruct((e, h), jnp.float32),
    )(ab, rad.reshape(e, 1), ea, w_r.reshape(1, h), W_at,
      b_e1.reshape(1, h), W_e2, b_e2.reshape(1, h))


def _sc_scatter(m2s, row3ds, n):
    nseg = len(m2s)
    e, h = m2s[0].shape
    ch = e // _NW
    nblk, sb = row3ds[0].shape[1], row3ds[0].shape[2]
    nb_n = n // sb
    nrounds = -(-nb_n // _NS)
    mesh = plsc.VectorSubcoreMesh(core_axis_name="c", subcore_axis_name="s")

    @functools.partial(
        pl.kernel,
        out_type=jax.ShapeDtypeStruct((_NC, n, h), jnp.float32),
        mesh=mesh,
        scratch_types=[
            pltpu.VMEM((nblk, sb), jnp.int32),
            pltpu.VMEM((sb, h), jnp.float32),
            pltpu.VMEM((sb, h), jnp.float32),
            pltpu.VMEM((sb, h), jnp.float32),
            pltpu.VMEM_SHARED((n, h), jnp.float32),
            pltpu.SemaphoreType.DMA,
            pltpu.SemaphoreType.DMA,
        ],
    )
    def k(*refs):
        m2_hbms = refs[:nseg]
        row3d_hbms = refs[nseg:2 * nseg]
        agg_hbm = refs[2 * nseg]
        idx_v, buf_m0, buf_m1, zbuf, agg_sh, sem_m0, sem_m1 = \
            refs[2 * nseg + 1:]
        cid = lax.axis_index("c")
        sid = lax.axis_index("s")
        wid = sid * _NC + cid
        base = wid * ch

        zeros16 = jnp.zeros((16,), jnp.float32)

        def z_body(i, carry):
            for j in range(h // 16):
                zbuf[i, pl.ds(j * 16, 16)] = zeros16
            return carry

        lax.fori_loop(0, sb, z_body, 0)
        for t in range(nrounds):
            blk = sid + t * _NS
            if nb_n % _NS == 0:
                pltpu.sync_copy(zbuf, agg_sh.at[pl.ds(blk * sb, sb)])
            else:
                @pl.when(blk < nb_n)
                def _():
                    pltpu.sync_copy(zbuf, agg_sh.at[pl.ds(blk * sb, sb)])
        plsc.subcore_barrier()

        for m2_hbm, row3d_hbm in zip(m2_hbms, row3d_hbms):
            pltpu.sync_copy(row3d_hbm.at[wid], idx_v)

            def s_body(i, carry):
                j0, j1 = 2 * i, 2 * i + 1
                c0 = pltpu.async_copy(
                    m2_hbm.at[pl.ds(base + j0 * sb, sb)], buf_m0, sem_m0)
                c1 = pltpu.async_copy(
                    m2_hbm.at[pl.ds(base + j1 * sb, sb)], buf_m1, sem_m1)
                c0.wait()
                pltpu.sync_copy(buf_m0, agg_sh.at[idx_v.at[j0]], add=True)
                c1.wait()
                pltpu.sync_copy(buf_m1, agg_sh.at[idx_v.at[j1]], add=True)
                return carry

            lax.fori_loop(0, nblk // 2, s_body, 0)
            for j in range(2 * (nblk // 2), nblk):
                pltpu.sync_copy(m2_hbm.at[pl.ds(base + j * sb, sb)], buf_m0)
                pltpu.sync_copy(buf_m0, agg_sh.at[idx_v.at[j]], add=True)
        plsc.subcore_barrier()
        for t in range(nrounds):
            blk = sid + t * _NS
            if nb_n % _NS == 0:
                pltpu.sync_copy(agg_sh.at[pl.ds(blk * sb, sb)],
                                agg_hbm.at[cid, pl.ds(blk * sb, sb)])
            else:
                @pl.when(blk < nb_n)
                def _():
                    pltpu.sync_copy(agg_sh.at[pl.ds(blk * sb, sb)],
                                    agg_hbm.at[cid, pl.ds(blk * sb, sb)])

    return k(*m2s, *row3ds)


def _node_body(h0_ref, h_ref, *rest):
    f32 = jnp.float32
    agg_refs = rest[:-8]
    (wnh_ref, wna_ref, b1_ref, wn2_ref, b2_ref, wo_ref, bo_ref,
     out_ref) = rest[-8:]
    agg = agg_refs[0][0] + agg_refs[0][1]
    for r in agg_refs[1:]:
        agg = agg + r[0] + r[1]
    t = _ssp(jnp.dot(h_ref[...], wnh_ref[...], preferred_element_type=f32)
             + jnp.dot(agg, wna_ref[...], preferred_element_type=f32)
             + b1_ref[...])
    hn = jnp.dot(t, wn2_ref[...], preferred_element_type=f32) + b2_ref[...]
    hn = jnp.dot(hn, wo_ref[...], preferred_element_type=f32) + bo_ref[...]
    h0 = h0_ref[...]
    out_ref[0] = h0
    out_ref[1] = h0 + hn


def _node(h0, hfeat, aggps, Wn1h, Wn1a, b_n1, W_n2, b_n2, W_out, b_out, bn):
    n, h = h0.shape
    grid = n // bn
    full = lambda s: pl.BlockSpec(s, lambda i: (0, 0))
    return pl.pallas_call(
        _node_body,
        grid=(grid,),
        in_specs=[
            pl.BlockSpec((bn, h), lambda i: (i, 0)),
            pl.BlockSpec((bn, h), lambda i: (i, 0)),
        ] + [pl.BlockSpec((_NC, bn, h), lambda i: (0, i, 0))] * len(aggps) + [
            full((h, h)), full((h, h)), full((1, h)),
            full((h, h)), full((1, h)), full((h, h)), full((1, h)),
        ],
        out_specs=pl.BlockSpec((2, bn, h), lambda i: (0, i, 0)),
        out_shape=jax.ShapeDtypeStruct((2, n, h), jnp.float32),
    )(h0, hfeat, *aggps, Wn1h, Wn1a, b_n1.reshape(1, h), W_n2,
      b_n2.reshape(1, h), W_out, b_out.reshape(1, h))


def kernel(x, edge_index, pos, edge_attr,
           W_emb, b_emb, W_in, b_in,
           W_e1, b_e1, W_e2, b_e2,
           W_n1, b_n1, W_n2, b_n2,
           W_c1, b_c1, W_c2,
           W_out, b_out):
    n, d = x.shape
    h = W_emb.shape[1]
    e = edge_index.shape[1]
    nseg = 2
    eseg = e // nseg
    sbg = 80
    sbs = 40

    row = edge_index[0].astype(jnp.int32)
    col = edge_index[1].astype(jnp.int32)
    px = pos[:, 0].astype(jnp.float32)
    py = pos[:, 1].astype(jnp.float32)
    pz = pos[:, 2].astype(jnp.float32)

    We1a = W_e1[:h]
    We1b = W_e1[h:2 * h]
    w_r = W_e1[2 * h]
    W_at = W_e1[2 * h + 1:]

    h0, hfeat, A, B = _prep(x, W_emb, b_emb, W_in, b_in, We1a, We1b, bn=1000)
    aggps = []
    for s in range(nseg):
        sl = slice(s * eseg, (s + 1) * eseg)
        row_s, col_s = row[sl], col[sl]
        ab, rad = _sc_gather(A, B, row_s, col_s, px, py, pz, sbg)
        m2 = _emlp(ab, rad, edge_attr[sl], w_r, W_at, b_e1, W_e2, b_e2,
                   be=2000)
        row3d = row_s.reshape(_NW, eseg // (_NW * sbs), sbs)
        aggps.append(_sc_scatter([m2], [row3d], n))
    return _node(h0, hfeat, aggps, W_n1[:h], W_n1[h:], b_n1, W_n2, b_n2,
                 W_out, b_out, bn=1000)


# speedup vs baseline: 1.5815x; 1.2964x over previous
import functools

import jax
import jax.numpy as jnp
from jax import lax
from jax.experimental import pallas as pl
from jax.experimental.pallas import tpu as pltpu
from jax.experimental.pallas import tpu_sc as plsc

_LOG2 = 0.6931471805599453
_NC = 2
_NS = 16
_NW = _NC * _NS


def _ssp(v):
    return jnp.maximum(v, 0.0) + jnp.log1p(jnp.exp(-jnp.abs(v))) - _LOG2


def _pack_bf16(a):
    hw = a.shape[1] // 2
    rn = jax.lax.bitcast_convert_type(
        a.astype(jnp.bfloat16).astype(jnp.float32), jnp.int32)
    lo = jax.lax.shift_right_logical(rn[:, :hw], jnp.int32(16))
    hi = rn[:, hw:] & jnp.int32(-65536)
    return lo | hi


def _unpack_sum(u):
    f32 = jnp.float32
    hw = u.shape[1] // 2
    lo = jax.lax.bitcast_convert_type(u << 16, f32)
    hi = jax.lax.bitcast_convert_type(u & jnp.int32(-65536), f32)
    return jnp.concatenate([lo[:, :hw] + lo[:, hw:], hi[:, :hw] + hi[:, hw:]],
                           axis=1)


def _prep_body(x_ref, we_ref, be_ref, wi_ref, bi_ref, wa_ref, wb_ref,
               h0_ref, h_ref, a_ref, b_ref):
    f32 = jnp.float32
    h0 = jnp.dot(x_ref[...], we_ref[...], preferred_element_type=f32) + be_ref[...]
    h = jnp.dot(h0, wi_ref[...], preferred_element_type=f32) + bi_ref[...]
    h0_ref[...] = h0
    h_ref[...] = h
    a = jnp.dot(h, wa_ref[...], preferred_element_type=f32)
    b = jnp.dot(h, wb_ref[...], preferred_element_type=f32)
    a_ref[...] = _pack_bf16(a)
    b_ref[...] = _pack_bf16(b)


def _prep(x, W_emb, b_emb, W_in, b_in, We1a, We1b, bn):
    n, d = x.shape
    h = W_emb.shape[1]
    grid = n // bn
    full = lambda s: pl.BlockSpec(s, lambda i: (0, 0))
    out_f = jax.ShapeDtypeStruct((n, h), jnp.float32)
    out_p = jax.ShapeDtypeStruct((n, h // 2), jnp.int32)
    return pl.pallas_call(
        _prep_body,
        grid=(grid,),
        in_specs=[
            pl.BlockSpec((bn, d), lambda i: (i, 0)),
            full((d, h)), full((1, h)), full((h, h)), full((1, h)),
            full((h, h)), full((h, h)),
        ],
        out_specs=[pl.BlockSpec((bn, h), lambda i: (i, 0)),
                   pl.BlockSpec((bn, h), lambda i: (i, 0)),
                   pl.BlockSpec((bn, h // 2), lambda i: (i, 0)),
                   pl.BlockSpec((bn, h // 2), lambda i: (i, 0))],
        out_shape=[out_f, out_f, out_p, out_p],
    )(x, W_emb, b_emb.reshape(1, h), W_in, b_in.reshape(1, h), We1a, We1b)


def _sc_gather(A, B, row, col, px, py, pz, sb):
    n, hw = A.shape
    e = row.shape[0]
    ch = e // _NW
    nblk = ch // sb
    btail = ch % sb != 0
    g16 = ch // 16
    gtail = ch % 16 != 0
    mesh = plsc.VectorSubcoreMesh(core_axis_name="c", subcore_axis_name="s")

    @functools.partial(
        pl.kernel,
        out_type=(jax.ShapeDtypeStruct((e, 2 * hw), jnp.int32),
                  jax.ShapeDtypeStruct((1, e), jnp.float32)),
        mesh=mesh,
        scratch_types=[
            pltpu.VMEM((ch,), jnp.int32),
            pltpu.VMEM((ch,), jnp.int32),
            pltpu.VMEM((n,), jnp.float32),
            pltpu.VMEM((n,), jnp.float32),
            pltpu.VMEM((n,), jnp.float32),
            pltpu.VMEM((ch,), jnp.float32),
            pltpu.VMEM((sb, hw), jnp.int32),
            pltpu.VMEM((sb, hw), jnp.int32),
            pltpu.VMEM((sb, hw), jnp.int32),
            pltpu.VMEM((sb, hw), jnp.int32),
            pltpu.SemaphoreType.DMA,
            pltpu.SemaphoreType.DMA,
            pltpu.SemaphoreType.DMA,
            pltpu.SemaphoreType.DMA,
        ],
        compiler_params=pltpu.CompilerParams(needs_layout_passes=False,
                                             use_tc_tiling_on_sc=False),
    )
    def k(a_hbm, b_hbm, row_hbm, col_hbm, px_hbm, py_hbm, pz_hbm,
          ab_hbm, rad_hbm,
          row_v, col_v, px_v, py_v, pz_v, rad_v,
          buf_a0, buf_b0, buf_a1, buf_b1, sem_a0, sem_b0, sem_a1, sem_b1):
        cid = lax.axis_index("c")
        sid = lax.axis_index("s")
        wid = sid * _NC + cid
        base = wid * ch
        pltpu.sync_copy(row_hbm.at[pl.ds(base, ch)], row_v)
        pltpu.sync_copy(col_hbm.at[pl.ds(base, ch)], col_v)
        pltpu.sync_copy(px_hbm, px_v)
        pltpu.sync_copy(py_hbm, py_v)
        pltpu.sync_copy(pz_hbm, pz_v)

        def rad_at(off):
            ir = row_v[pl.ds(off, 16)]
            ic = col_v[pl.ds(off, 16)]
            dx = plsc.load_gather(px_v, [ir]) - plsc.load_gather(px_v, [ic])
            dy = plsc.load_gather(py_v, [ir]) - plsc.load_gather(py_v, [ic])
            dz = plsc.load_gather(pz_v, [ir]) - plsc.load_gather(pz_v, [ic])
            rad_v[pl.ds(off, 16)] = dx * dx + dy * dy + dz * dz

        def rad_body(i, carry):
            rad_at(i * 16)
            return carry

        lax.fori_loop(0, g16, rad_body, 0)
        if gtail:
            rad_at(ch - 16)
        pltpu.sync_copy(rad_v, rad_hbm.at[0, pl.ds(base, ch)])

        def issue(off, buf_a, buf_b, sem_a, sem_b):
            ca = pltpu.async_copy(a_hbm.at[row_v.at[pl.ds(off, sb)]], buf_a, sem_a)
            cb = pltpu.async_copy(b_hbm.at[col_v.at[pl.ds(off, sb)]], buf_b, sem_b)
            return ca, cb

        def drain(off, ca, cb, buf_a, buf_b):
            ca.wait()
            cb.wait()
            pltpu.sync_copy(buf_a, ab_hbm.at[pl.ds(base + off, sb), pl.ds(0, hw)])
            pltpu.sync_copy(buf_b, ab_hbm.at[pl.ds(base + off, sb), pl.ds(hw, hw)])

        npair = nblk // 2

        def blk_body(i, carry):
            o0 = (2 * i) * sb
            o1 = (2 * i + 1) * sb
            c0 = issue(o0, buf_a0, buf_b0, sem_a0, sem_b0)
            c1 = issue(o1, buf_a1, buf_b1, sem_a1, sem_b1)
            drain(o0, *c0, buf_a0, buf_b0)
            drain(o1, *c1, buf_a1, buf_b1)
            return carry

        lax.fori_loop(0, npair, blk_body, 0)
        for j in range(2 * npair, nblk):
            c0 = issue(j * sb, buf_a0, buf_b0, sem_a0, sem_b0)
            drain(j * sb, *c0, buf_a0, buf_b0)
        if btail:
            c0 = issue(ch - sb, buf_a0, buf_b0, sem_a0, sem_b0)
            drain(ch - sb, *c0, buf_a0, buf_b0)

    return k(A, B, row, col, px, py, pz)


_DN_T = (((0,), (0,)), ((), ()))


def _emlp_body(ab_ref, rad_ref, eat_ref, wr_ref, wat_ref, b1_ref,
               we2_ref, b2_ref, m2_ref):
    f32 = jnp.float32
    m1 = (_unpack_sum(ab_ref[...])
          + jax.lax.dot_general(rad_ref[...], wr_ref[...], _DN_T,
                                preferred_element_type=f32)
          + jax.lax.dot_general(eat_ref[...], wat_ref[...], _DN_T,
                                preferred_element_type=f32)
          + b1_ref[...])
    em = _ssp(m1)
    m2_ref[...] = _ssp(jnp.dot(em, we2_ref[...], preferred_element_type=f32)
                       + b2_ref[...])


def _emlp(ab, rad, eat, w_r, W_at, b_e1, W_e2, b_e2, be):
    e, h = ab.shape
    enf = eat.shape[0]
    grid = e // be
    full = lambda s: pl.BlockSpec(s, lambda i: (0, 0))
    return pl.pallas_call(
        _emlp_body,
        grid=(grid,),
        in_specs=[
            pl.BlockSpec((be, h), lambda i: (i, 0)),
            pl.BlockSpec((1, be), lambda i: (0, i)),
            pl.BlockSpec((enf, be), lambda i: (0, i)),
            full((1, h)), full((enf, h)), full((1, h)),
            full((h, h)), full((1, h)),
        ],
        out_specs=pl.BlockSpec((be, h), lambda i: (i, 0)),
        out_shape=jax.ShapeDtypeStruct((e, h), jnp.float32),
    )(ab, rad, eat, w_r.reshape(1, h), W_at,
      b_e1.reshape(1, h), W_e2, b_e2.reshape(1, h))


def _sc_scatter(m2s, row3ds, n):
    nseg = len(m2s)
    e, h = m2s[0].shape
    ch = e // _NW
    nblk, sb = row3ds[0].shape[1], row3ds[0].shape[2]
    nb_n = n // sb
    nrounds = -(-nb_n // _NS)
    mesh = plsc.VectorSubcoreMesh(core_axis_name="c", subcore_axis_name="s")

    @functools.partial(
        pl.kernel,
        out_type=jax.ShapeDtypeStruct((_NC, n, h), jnp.float32),
        mesh=mesh,
        scratch_types=[
            pltpu.VMEM((nblk, sb), jnp.int32),
            pltpu.VMEM((sb, h), jnp.float32),
            pltpu.VMEM((sb, h), jnp.float32),
            pltpu.VMEM((sb, h), jnp.float32),
            pltpu.VMEM_SHARED((n, h), jnp.float32),
            pltpu.SemaphoreType.DMA,
            pltpu.SemaphoreType.DMA,
        ],
    )
    def k(*refs):
        m2_hbms = refs[:nseg]
        row3d_hbms = refs[nseg:2 * nseg]
        agg_hbm = refs[2 * nseg]
        idx_v, buf_m0, buf_m1, zbuf, agg_sh, sem_m0, sem_m1 = \
            refs[2 * nseg + 1:]
        cid = lax.axis_index("c")
        sid = lax.axis_index("s")
        wid = sid * _NC + cid
        base = wid * ch

        zeros16 = jnp.zeros((16,), jnp.float32)

        def z_body(i, carry):
            for j in range(h // 16):
                zbuf[i, pl.ds(j * 16, 16)] = zeros16
            return carry

        lax.fori_loop(0, sb, z_body, 0)
        for t in range(nrounds):
            blk = sid + t * _NS
            if nb_n % _NS == 0:
                pltpu.sync_copy(zbuf, agg_sh.at[pl.ds(blk * sb, sb)])
            else:
                @pl.when(blk < nb_n)
                def _():
                    pltpu.sync_copy(zbuf, agg_sh.at[pl.ds(blk * sb, sb)])
        plsc.subcore_barrier()

        for m2_hbm, row3d_hbm in zip(m2_hbms, row3d_hbms):
            pltpu.sync_copy(row3d_hbm.at[wid], idx_v)

            def s_body(i, carry):
                j0, j1 = 2 * i, 2 * i + 1
                c0 = pltpu.async_copy(
                    m2_hbm.at[pl.ds(base + j0 * sb, sb)], buf_m0, sem_m0)
                c1 = pltpu.async_copy(
                    m2_hbm.at[pl.ds(base + j1 * sb, sb)], buf_m1, sem_m1)
                c0.wait()
                pltpu.sync_copy(buf_m0, agg_sh.at[idx_v.at[j0]], add=True)
                c1.wait()
                pltpu.sync_copy(buf_m1, agg_sh.at[idx_v.at[j1]], add=True)
                return carry

            lax.fori_loop(0, nblk // 2, s_body, 0)
            for j in range(2 * (nblk // 2), nblk):
                pltpu.sync_copy(m2_hbm.at[pl.ds(base + j * sb, sb)], buf_m0)
                pltpu.sync_copy(buf_m0, agg_sh.at[idx_v.at[j]], add=True)
        plsc.subcore_barrier()
        for t in range(nrounds):
            blk = sid + t * _NS
            if nb_n % _NS == 0:
                pltpu.sync_copy(agg_sh.at[pl.ds(blk * sb, sb)],
                                agg_hbm.at[cid, pl.ds(blk * sb, sb)])
            else:
                @pl.when(blk < nb_n)
                def _():
                    pltpu.sync_copy(agg_sh.at[pl.ds(blk * sb, sb)],
                                    agg_hbm.at[cid, pl.ds(blk * sb, sb)])

    return k(*m2s, *row3ds)


def _node_body(h0_ref, h_ref, *rest):
    f32 = jnp.float32
    agg_refs = rest[:-8]
    (wnh_ref, wna_ref, b1_ref, wn2_ref, b2_ref, wo_ref, bo_ref,
     out_ref) = rest[-8:]
    agg = agg_refs[0][0] + agg_refs[0][1]
    for r in agg_refs[1:]:
        agg = agg + r[0] + r[1]
    t = _ssp(jnp.dot(h_ref[...], wnh_ref[...], preferred_element_type=f32)
             + jnp.dot(agg, wna_ref[...], preferred_element_type=f32)
             + b1_ref[...])
    hn = jnp.dot(t, wn2_ref[...], preferred_element_type=f32) + b2_ref[...]
    hn = jnp.dot(hn, wo_ref[...], preferred_element_type=f32) + bo_ref[...]
    h0 = h0_ref[...]
    out_ref[0] = h0
    out_ref[1] = h0 + hn


def _node(h0, hfeat, aggps, Wn1h, Wn1a, b_n1, W_n2, b_n2, W_out, b_out, bn):
    n, h = h0.shape
    grid = n // bn
    full = lambda s: pl.BlockSpec(s, lambda i: (0, 0))
    return pl.pallas_call(
        _node_body,
        grid=(grid,),
        in_specs=[
            pl.BlockSpec((bn, h), lambda i: (i, 0)),
            pl.BlockSpec((bn, h), lambda i: (i, 0)),
        ] + [pl.BlockSpec((_NC, bn, h), lambda i: (0, i, 0))] * len(aggps) + [
            full((h, h)), full((h, h)), full((1, h)),
            full((h, h)), full((1, h)), full((h, h)), full((1, h)),
        ],
        out_specs=pl.BlockSpec((2, bn, h), lambda i: (0, i, 0)),
        out_shape=jax.ShapeDtypeStruct((2, n, h), jnp.float32),
    )(h0, hfeat, *aggps, Wn1h, Wn1a, b_n1.reshape(1, h), W_n2,
      b_n2.reshape(1, h), W_out, b_out.reshape(1, h))


def kernel(x, edge_index, pos, edge_attr,
           W_emb, b_emb, W_in, b_in,
           W_e1, b_e1, W_e2, b_e2,
           W_n1, b_n1, W_n2, b_n2,
           W_c1, b_c1, W_c2,
           W_out, b_out):
    n, d = x.shape
    h = W_emb.shape[1]
    e = edge_index.shape[1]
    nseg = 2
    eseg = e // nseg
    sbg = 80
    sbs = 40

    row = edge_index[0].astype(jnp.int32)
    col = edge_index[1].astype(jnp.int32)
    px = pos[:, 0].astype(jnp.float32)
    py = pos[:, 1].astype(jnp.float32)
    pz = pos[:, 2].astype(jnp.float32)

    We1a = W_e1[:h]
    We1b = W_e1[h:2 * h]
    w_r = W_e1[2 * h]
    W_at = W_e1[2 * h + 1:]

    h0, hfeat, A, B = _prep(x, W_emb, b_emb, W_in, b_in, We1a, We1b, bn=1000)
    eaT = edge_attr.T
    aggps = []
    for s in range(nseg):
        sl = slice(s * eseg, (s + 1) * eseg)
        row_s, col_s = row[sl], col[sl]
        ab, rad = _sc_gather(A, B, row_s, col_s, px, py, pz, sbg)
        m2 = _emlp(ab, rad, eaT[:, sl], w_r, W_at, b_e1, W_e2, b_e2,
                   be=3200)
        row3d = row_s.reshape(_NW, eseg // (_NW * sbs), sbs)
        aggps.append(_sc_scatter([m2], [row3d], n))
    return _node(h0, hfeat, aggps, W_n1[:h], W_n1[h:], b_n1, W_n2, b_n2,
                 W_out, b_out, bn=1000)


# speedup vs baseline: 1.7723x; 1.1206x over previous
import functools

import jax
import jax.numpy as jnp
from jax import lax
from jax.experimental import pallas as pl
from jax.experimental.pallas import tpu as pltpu
from jax.experimental.pallas import tpu_sc as plsc

_LOG2 = 0.6931471805599453
_NC = 2
_NS = 16
_NW = _NC * _NS


def _ssp(v):
    return jnp.maximum(v, 0.0) + jnp.log1p(jnp.exp(-jnp.abs(v))) - _LOG2


def _pack_bf16(a):
    hw = a.shape[1] // 2
    rn = jax.lax.bitcast_convert_type(
        a.astype(jnp.bfloat16).astype(jnp.float32), jnp.int32)
    lo = jax.lax.shift_right_logical(rn[:, :hw], jnp.int32(16))
    hi = rn[:, hw:] & jnp.int32(-65536)
    return lo | hi


def _unpack_sum(u):
    f32 = jnp.float32
    hw = u.shape[1] // 2
    lo = jax.lax.bitcast_convert_type(u << 16, f32)
    hi = jax.lax.bitcast_convert_type(u & jnp.int32(-65536), f32)
    return jnp.concatenate([lo[:, :hw] + lo[:, hw:], hi[:, :hw] + hi[:, hw:]],
                           axis=1)


def _prep_body(x_ref, we_ref, be_ref, wi_ref, bi_ref, wa_ref, wb_ref,
               h0_ref, h_ref, a_ref, b_ref):
    f32 = jnp.float32
    h0 = jnp.dot(x_ref[...], we_ref[...], preferred_element_type=f32) + be_ref[...]
    h = jnp.dot(h0, wi_ref[...], preferred_element_type=f32) + bi_ref[...]
    h0_ref[...] = h0
    h_ref[...] = h
    a = jnp.dot(h, wa_ref[...], preferred_element_type=f32)
    b = jnp.dot(h, wb_ref[...], preferred_element_type=f32)
    a_ref[...] = _pack_bf16(a)
    b_ref[...] = _pack_bf16(b)


def _prep(x, W_emb, b_emb, W_in, b_in, We1a, We1b, bn):
    n, d = x.shape
    h = W_emb.shape[1]
    grid = n // bn
    full = lambda s: pl.BlockSpec(s, lambda i: (0, 0))
    out_f = jax.ShapeDtypeStruct((n, h), jnp.float32)
    out_p = jax.ShapeDtypeStruct((n, h // 2), jnp.int32)
    return pl.pallas_call(
        _prep_body,
        grid=(grid,),
        in_specs=[
            pl.BlockSpec((bn, d), lambda i: (i, 0)),
            full((d, h)), full((1, h)), full((h, h)), full((1, h)),
            full((h, h)), full((h, h)),
        ],
        out_specs=[pl.BlockSpec((bn, h), lambda i: (i, 0)),
                   pl.BlockSpec((bn, h), lambda i: (i, 0)),
                   pl.BlockSpec((bn, h // 2), lambda i: (i, 0)),
                   pl.BlockSpec((bn, h // 2), lambda i: (i, 0))],
        out_shape=[out_f, out_f, out_p, out_p],
    )(x, W_emb, b_emb.reshape(1, h), W_in, b_in.reshape(1, h), We1a, We1b)


def _sc_gather(A, B, row, col, px, py, pz, sb):
    n, hw = A.shape
    e = row.shape[0]
    ch = e // _NW
    nblk = ch // sb
    btail = ch % sb != 0
    g16 = ch // 16
    gtail = ch % 16 != 0
    mesh = plsc.VectorSubcoreMesh(core_axis_name="c", subcore_axis_name="s")

    @functools.partial(
        pl.kernel,
        out_type=(jax.ShapeDtypeStruct((e, 2 * hw), jnp.int32),
                  jax.ShapeDtypeStruct((1, e), jnp.float32)),
        mesh=mesh,
        scratch_types=[
            pltpu.VMEM((ch,), jnp.int32),
            pltpu.VMEM((ch,), jnp.int32),
            pltpu.VMEM((n,), jnp.float32),
            pltpu.VMEM((n,), jnp.float32),
            pltpu.VMEM((n,), jnp.float32),
            pltpu.VMEM((ch,), jnp.float32),
            pltpu.VMEM((sb, hw), jnp.int32),
            pltpu.VMEM((sb, hw), jnp.int32),
            pltpu.VMEM((sb, hw), jnp.int32),
            pltpu.VMEM((sb, hw), jnp.int32),
            pltpu.SemaphoreType.DMA,
            pltpu.SemaphoreType.DMA,
            pltpu.SemaphoreType.DMA,
            pltpu.SemaphoreType.DMA,
        ],
        compiler_params=pltpu.CompilerParams(needs_layout_passes=False,
                                             use_tc_tiling_on_sc=False),
    )
    def k(a_hbm, b_hbm, row_hbm, col_hbm, px_hbm, py_hbm, pz_hbm,
          ab_hbm, rad_hbm,
          row_v, col_v, px_v, py_v, pz_v, rad_v,
          buf_a0, buf_b0, buf_a1, buf_b1, sem_a0, sem_b0, sem_a1, sem_b1):
        cid = lax.axis_index("c")
        sid = lax.axis_index("s")
        wid = sid * _NC + cid
        base = wid * ch
        pltpu.sync_copy(row_hbm.at[pl.ds(base, ch)], row_v)
        pltpu.sync_copy(col_hbm.at[pl.ds(base, ch)], col_v)
        pltpu.sync_copy(px_hbm, px_v)
        pltpu.sync_copy(py_hbm, py_v)
        pltpu.sync_copy(pz_hbm, pz_v)

        def rad_at(off):
            ir = row_v[pl.ds(off, 16)]
            ic = col_v[pl.ds(off, 16)]
            dx = plsc.load_gather(px_v, [ir]) - plsc.load_gather(px_v, [ic])
            dy = plsc.load_gather(py_v, [ir]) - plsc.load_gather(py_v, [ic])
            dz = plsc.load_gather(pz_v, [ir]) - plsc.load_gather(pz_v, [ic])
            rad_v[pl.ds(off, 16)] = dx * dx + dy * dy + dz * dz

        def rad_body(i, carry):
            rad_at(i * 16)
            return carry

        lax.fori_loop(0, g16, rad_body, 0)
        if gtail:
            rad_at(ch - 16)
        pltpu.sync_copy(rad_v, rad_hbm.at[0, pl.ds(base, ch)])

        def issue(off, buf_a, buf_b, sem_a, sem_b):
            ca = pltpu.async_copy(a_hbm.at[row_v.at[pl.ds(off, sb)]], buf_a, sem_a)
            cb = pltpu.async_copy(b_hbm.at[col_v.at[pl.ds(off, sb)]], buf_b, sem_b)
            return ca, cb

        def drain(off, ca, cb, buf_a, buf_b):
            ca.wait()
            cb.wait()
            pltpu.sync_copy(buf_a, ab_hbm.at[pl.ds(base + off, sb), pl.ds(0, hw)])
            pltpu.sync_copy(buf_b, ab_hbm.at[pl.ds(base + off, sb), pl.ds(hw, hw)])

        npair = nblk // 2

        def blk_body(i, carry):
            o0 = (2 * i) * sb
            o1 = (2 * i + 1) * sb
            c0 = issue(o0, buf_a0, buf_b0, sem_a0, sem_b0)
            c1 = issue(o1, buf_a1, buf_b1, sem_a1, sem_b1)
            drain(o0, *c0, buf_a0, buf_b0)
            drain(o1, *c1, buf_a1, buf_b1)
            return carry

        lax.fori_loop(0, npair, blk_body, 0)
        for j in range(2 * npair, nblk):
            c0 = issue(j * sb, buf_a0, buf_b0, sem_a0, sem_b0)
            drain(j * sb, *c0, buf_a0, buf_b0)
        if btail:
            c0 = issue(ch - sb, buf_a0, buf_b0, sem_a0, sem_b0)
            drain(ch - sb, *c0, buf_a0, buf_b0)

    return k(A, B, row, col, px, py, pz)


_DN_T = (((0,), (0,)), ((), ()))


def _emlp_body(ab_ref, rad_ref, eat_ref, wr_ref, wat_ref, b1_ref,
               we2_ref, b2_ref, m2_ref):
    f32 = jnp.float32
    m1 = (_unpack_sum(ab_ref[...])
          + jax.lax.dot_general(rad_ref[...], wr_ref[...], _DN_T,
                                preferred_element_type=f32)
          + jax.lax.dot_general(eat_ref[...], wat_ref[...], _DN_T,
                                preferred_element_type=f32)
          + b1_ref[...])
    em = _ssp(m1).astype(jnp.bfloat16)
    m2_ref[...] = _ssp(jnp.dot(em, we2_ref[...], preferred_element_type=f32)
                       + b2_ref[...])


def _emlp(ab, rad, eat, w_r, W_at, b_e1, W_e2, b_e2, be):
    e, h = ab.shape
    enf = eat.shape[0]
    grid = e // be
    full = lambda s: pl.BlockSpec(s, lambda i: (0, 0))
    return pl.pallas_call(
        _emlp_body,
        grid=(grid,),
        in_specs=[
            pl.BlockSpec((be, h), lambda i: (i, 0)),
            pl.BlockSpec((1, be), lambda i: (0, i)),
            pl.BlockSpec((enf, be), lambda i: (0, i)),
            full((1, h)), full((enf, h)), full((1, h)),
            full((h, h)), full((1, h)),
        ],
        out_specs=pl.BlockSpec((be, h), lambda i: (i, 0)),
        out_shape=jax.ShapeDtypeStruct((e, h), jnp.float32),
    )(ab, rad, eat, w_r.reshape(1, h), W_at,
      b_e1.reshape(1, h), W_e2.astype(jnp.bfloat16), b_e2.reshape(1, h))


def _sc_scatter(m2s, row3ds, n):
    nseg = len(m2s)
    e, h = m2s[0].shape
    ch = e // _NW
    nblk, sb = row3ds[0].shape[1], row3ds[0].shape[2]
    nb_n = n // sb
    nrounds = -(-nb_n // _NS)
    mesh = plsc.VectorSubcoreMesh(core_axis_name="c", subcore_axis_name="s")

    @functools.partial(
        pl.kernel,
        out_type=jax.ShapeDtypeStruct((_NC, n, h), jnp.float32),
        mesh=mesh,
        scratch_types=[
            pltpu.VMEM((nblk, sb), jnp.int32),
            pltpu.VMEM((sb, h), jnp.float32),
            pltpu.VMEM((sb, h), jnp.float32),
            pltpu.VMEM((sb, h), jnp.float32),
            pltpu.VMEM_SHARED((n, h), jnp.float32),
            pltpu.SemaphoreType.DMA,
            pltpu.SemaphoreType.DMA,
        ],
    )
    def k(*refs):
        m2_hbms = refs[:nseg]
        row3d_hbms = refs[nseg:2 * nseg]
        agg_hbm = refs[2 * nseg]
        idx_v, buf_m0, buf_m1, zbuf, agg_sh, sem_m0, sem_m1 = \
            refs[2 * nseg + 1:]
        cid = lax.axis_index("c")
        sid = lax.axis_index("s")
        wid = sid * _NC + cid
        base = wid * ch

        zeros16 = jnp.zeros((16,), jnp.float32)

        def z_body(i, carry):
            for j in range(h // 16):
                zbuf[i, pl.ds(j * 16, 16)] = zeros16
            return carry

        lax.fori_loop(0, sb, z_body, 0)
        for t in range(nrounds):
            blk = sid + t * _NS
            if nb_n % _NS == 0:
                pltpu.sync_copy(zbuf, agg_sh.at[pl.ds(blk * sb, sb)])
            else:
                @pl.when(blk < nb_n)
                def _():
                    pltpu.sync_copy(zbuf, agg_sh.at[pl.ds(blk * sb, sb)])
        plsc.subcore_barrier()

        for m2_hbm, row3d_hbm in zip(m2_hbms, row3d_hbms):
            pltpu.sync_copy(row3d_hbm.at[wid], idx_v)

            def s_body(i, carry):
                j0, j1 = 2 * i, 2 * i + 1
                c0 = pltpu.async_copy(
                    m2_hbm.at[pl.ds(base + j0 * sb, sb)], buf_m0, sem_m0)
                c1 = pltpu.async_copy(
                    m2_hbm.at[pl.ds(base + j1 * sb, sb)], buf_m1, sem_m1)
                c0.wait()
                pltpu.sync_copy(buf_m0, agg_sh.at[idx_v.at[j0]], add=True)
                c1.wait()
                pltpu.sync_copy(buf_m1, agg_sh.at[idx_v.at[j1]], add=True)
                return carry

            lax.fori_loop(0, nblk // 2, s_body, 0)
            for j in range(2 * (nblk // 2), nblk):
                pltpu.sync_copy(m2_hbm.at[pl.ds(base + j * sb, sb)], buf_m0)
                pltpu.sync_copy(buf_m0, agg_sh.at[idx_v.at[j]], add=True)
        plsc.subcore_barrier()
        for t in range(nrounds):
            blk = sid + t * _NS
            if nb_n % _NS == 0:
                pltpu.sync_copy(agg_sh.at[pl.ds(blk * sb, sb)],
                                agg_hbm.at[cid, pl.ds(blk * sb, sb)])
            else:
                @pl.when(blk < nb_n)
                def _():
                    pltpu.sync_copy(agg_sh.at[pl.ds(blk * sb, sb)],
                                    agg_hbm.at[cid, pl.ds(blk * sb, sb)])

    return k(*m2s, *row3ds)


def _node_body(h0_ref, h_ref, *rest):
    f32 = jnp.float32
    agg_refs = rest[:-8]
    (wnh_ref, wna_ref, b1_ref, wn2_ref, b2_ref, wo_ref, bo_ref,
     out_ref) = rest[-8:]
    agg = agg_refs[0][0] + agg_refs[0][1]
    for r in agg_refs[1:]:
        agg = agg + r[0] + r[1]
    t = _ssp(jnp.dot(h_ref[...], wnh_ref[...], preferred_element_type=f32)
             + jnp.dot(agg, wna_ref[...], preferred_element_type=f32)
             + b1_ref[...])
    hn = jnp.dot(t, wn2_ref[...], preferred_element_type=f32) + b2_ref[...]
    hn = jnp.dot(hn, wo_ref[...], preferred_element_type=f32) + bo_ref[...]
    h0 = h0_ref[...]
    out_ref[0] = h0
    out_ref[1] = h0 + hn


def _node(h0, hfeat, aggps, Wn1h, Wn1a, b_n1, W_n2, b_n2, W_out, b_out, bn):
    n, h = h0.shape
    grid = n // bn
    full = lambda s: pl.BlockSpec(s, lambda i: (0, 0))
    return pl.pallas_call(
        _node_body,
        grid=(grid,),
        in_specs=[
            pl.BlockSpec((bn, h), lambda i: (i, 0)),
            pl.BlockSpec((bn, h), lambda i: (i, 0)),
        ] + [pl.BlockSpec((_NC, bn, h), lambda i: (0, i, 0))] * len(aggps) + [
            full((h, h)), full((h, h)), full((1, h)),
            full((h, h)), full((1, h)), full((h, h)), full((1, h)),
        ],
        out_specs=pl.BlockSpec((2, bn, h), lambda i: (0, i, 0)),
        out_shape=jax.ShapeDtypeStruct((2, n, h), jnp.float32),
    )(h0, hfeat, *aggps, Wn1h, Wn1a, b_n1.reshape(1, h), W_n2,
      b_n2.reshape(1, h), W_out, b_out.reshape(1, h))


def kernel(x, edge_index, pos, edge_attr,
           W_emb, b_emb, W_in, b_in,
           W_e1, b_e1, W_e2, b_e2,
           W_n1, b_n1, W_n2, b_n2,
           W_c1, b_c1, W_c2,
           W_out, b_out):
    n, d = x.shape
    h = W_emb.shape[1]
    e = edge_index.shape[1]
    unit = e // 250
    segs = [100 * unit, 100 * unit, 50 * unit]
    sbg = 80
    sbs = 40

    row = edge_index[0].astype(jnp.int32)
    col = edge_index[1].astype(jnp.int32)
    px = pos[:, 0].astype(jnp.float32)
    py = pos[:, 1].astype(jnp.float32)
    pz = pos[:, 2].astype(jnp.float32)

    We1a = W_e1[:h]
    We1b = W_e1[h:2 * h]
    w_r = W_e1[2 * h]
    W_at = W_e1[2 * h + 1:]

    h0, hfeat, A, B = _prep(x, W_emb, b_emb, W_in, b_in, We1a, We1b, bn=1000)
    eaT = edge_attr.T
    aggps = []
    off = 0
    for eseg in segs:
        sl = slice(off, off + eseg)
        off += eseg
        row_s, col_s = row[sl], col[sl]
        ab, rad = _sc_gather(A, B, row_s, col_s, px, py, pz, sbg)
        m2 = _emlp(ab, rad, eaT[:, sl], w_r, W_at, b_e1, W_e2, b_e2,
                   be=3200)
        row3d = row_s.reshape(_NW, eseg // (_NW * sbs), sbs)
        aggps.append(_sc_scatter([m2], [row3d], n))
    return _node(h0, hfeat, aggps, W_n1[:h], W_n1[h:], b_n1, W_n2, b_n2,
                 W_out, b_out, bn=1000)


# speedup vs baseline: 1.7981x; 1.0145x over previous
import functools

import jax
import jax.numpy as jnp
from jax import lax
from jax.experimental import pallas as pl
from jax.experimental.pallas import tpu as pltpu
from jax.experimental.pallas import tpu_sc as plsc

_LOG2 = 0.6931471805599453
_NC = 2
_NS = 16
_NW = _NC * _NS


def _ssp(v):
    return jnp.maximum(v, 0.0) + jnp.log1p(jnp.exp(-jnp.abs(v))) - _LOG2


def _pack_bf16(a):
    hw = a.shape[1] // 2
    rn = jax.lax.bitcast_convert_type(
        a.astype(jnp.bfloat16).astype(jnp.float32), jnp.int32)
    lo = jax.lax.shift_right_logical(rn[:, :hw], jnp.int32(16))
    hi = rn[:, hw:] & jnp.int32(-65536)
    return lo | hi


def _unpack_sum(u):
    f32 = jnp.float32
    hw = u.shape[1] // 2
    lo = jax.lax.bitcast_convert_type(u << 16, f32)
    hi = jax.lax.bitcast_convert_type(u & jnp.int32(-65536), f32)
    return jnp.concatenate([lo[:, :hw] + lo[:, hw:], hi[:, :hw] + hi[:, hw:]],
                           axis=1)


def _prep_body(x_ref, we_ref, be_ref, wi_ref, bi_ref, wa_ref, wb_ref,
               h0_ref, h_ref, a_ref, b_ref):
    f32 = jnp.float32
    h0 = jnp.dot(x_ref[...], we_ref[...], preferred_element_type=f32) + be_ref[...]
    h = jnp.dot(h0, wi_ref[...], preferred_element_type=f32) + bi_ref[...]
    h0_ref[...] = h0
    h_ref[...] = h
    a = jnp.dot(h, wa_ref[...], preferred_element_type=f32)
    b = jnp.dot(h, wb_ref[...], preferred_element_type=f32)
    a_ref[...] = _pack_bf16(a)
    b_ref[...] = _pack_bf16(b)


def _prep(x, W_emb, b_emb, W_in, b_in, We1a, We1b, bn):
    n, d = x.shape
    h = W_emb.shape[1]
    grid = n // bn
    full = lambda s: pl.BlockSpec(s, lambda i: (0, 0))
    out_f = jax.ShapeDtypeStruct((n, h), jnp.float32)
    out_p = jax.ShapeDtypeStruct((n, h // 2), jnp.int32)
    return pl.pallas_call(
        _prep_body,
        grid=(grid,),
        in_specs=[
            pl.BlockSpec((bn, d), lambda i: (i, 0)),
            full((d, h)), full((1, h)), full((h, h)), full((1, h)),
            full((h, h)), full((h, h)),
        ],
        out_specs=[pl.BlockSpec((bn, h), lambda i: (i, 0)),
                   pl.BlockSpec((bn, h), lambda i: (i, 0)),
                   pl.BlockSpec((bn, h // 2), lambda i: (i, 0)),
                   pl.BlockSpec((bn, h // 2), lambda i: (i, 0))],
        out_shape=[out_f, out_f, out_p, out_p],
    )(x, W_emb, b_emb.reshape(1, h), W_in, b_in.reshape(1, h), We1a, We1b)


def _sc_gather(A, B, row, col, px, py, pz, sb):
    n, hw = A.shape
    e = row.shape[0]
    ch = e // _NW
    nblk = ch // sb
    btail = ch % sb != 0
    g16 = ch // 16
    gtail = ch % 16 != 0
    mesh = plsc.VectorSubcoreMesh(core_axis_name="c", subcore_axis_name="s")

    @functools.partial(
        pl.kernel,
        out_type=(jax.ShapeDtypeStruct((e, 2 * hw), jnp.int32),
                  jax.ShapeDtypeStruct((1, e), jnp.float32)),
        mesh=mesh,
        scratch_types=[
            pltpu.VMEM((ch,), jnp.int32),
            pltpu.VMEM((ch,), jnp.int32),
            pltpu.VMEM((n,), jnp.float32),
            pltpu.VMEM((n,), jnp.float32),
            pltpu.VMEM((n,), jnp.float32),
            pltpu.VMEM((ch,), jnp.float32),
            pltpu.VMEM((sb, hw), jnp.int32),
            pltpu.VMEM((sb, hw), jnp.int32),
            pltpu.VMEM((sb, hw), jnp.int32),
            pltpu.VMEM((sb, hw), jnp.int32),
            pltpu.SemaphoreType.DMA,
            pltpu.SemaphoreType.DMA,
            pltpu.SemaphoreType.DMA,
            pltpu.SemaphoreType.DMA,
        ],
        compiler_params=pltpu.CompilerParams(needs_layout_passes=False,
                                             use_tc_tiling_on_sc=False),
    )
    def k(a_hbm, b_hbm, row_hbm, col_hbm, px_hbm, py_hbm, pz_hbm,
          ab_hbm, rad_hbm,
          row_v, col_v, px_v, py_v, pz_v, rad_v,
          buf_a0, buf_b0, buf_a1, buf_b1, sem_a0, sem_b0, sem_a1, sem_b1):
        cid = lax.axis_index("c")
        sid = lax.axis_index("s")
        wid = sid * _NC + cid
        base = wid * ch
        pltpu.sync_copy(row_hbm.at[pl.ds(base, ch)], row_v)
        pltpu.sync_copy(col_hbm.at[pl.ds(base, ch)], col_v)
        pltpu.sync_copy(px_hbm, px_v)
        pltpu.sync_copy(py_hbm, py_v)
        pltpu.sync_copy(pz_hbm, pz_v)

        def rad_at(off):
            ir = row_v[pl.ds(off, 16)]
            ic = col_v[pl.ds(off, 16)]
            dx = plsc.load_gather(px_v, [ir]) - plsc.load_gather(px_v, [ic])
            dy = plsc.load_gather(py_v, [ir]) - plsc.load_gather(py_v, [ic])
            dz = plsc.load_gather(pz_v, [ir]) - plsc.load_gather(pz_v, [ic])
            rad_v[pl.ds(off, 16)] = dx * dx + dy * dy + dz * dz

        def rad_body(i, carry):
            rad_at(i * 16)
            return carry

        lax.fori_loop(0, g16, rad_body, 0)
        if gtail:
            rad_at(ch - 16)
        pltpu.sync_copy(rad_v, rad_hbm.at[0, pl.ds(base, ch)])

        def issue(off, buf_a, buf_b, sem_a, sem_b):
            ca = pltpu.async_copy(a_hbm.at[row_v.at[pl.ds(off, sb)]], buf_a, sem_a)
            cb = pltpu.async_copy(b_hbm.at[col_v.at[pl.ds(off, sb)]], buf_b, sem_b)
            return ca, cb

        def drain(off, ca, cb, buf_a, buf_b):
            ca.wait()
            cb.wait()
            pltpu.sync_copy(buf_a, ab_hbm.at[pl.ds(base + off, sb), pl.ds(0, hw)])
            pltpu.sync_copy(buf_b, ab_hbm.at[pl.ds(base + off, sb), pl.ds(hw, hw)])

        npair = nblk // 2

        def blk_body(i, carry):
            o0 = (2 * i) * sb
            o1 = (2 * i + 1) * sb
            c0 = issue(o0, buf_a0, buf_b0, sem_a0, sem_b0)
            c1 = issue(o1, buf_a1, buf_b1, sem_a1, sem_b1)
            drain(o0, *c0, buf_a0, buf_b0)
            drain(o1, *c1, buf_a1, buf_b1)
            return carry

        lax.fori_loop(0, npair, blk_body, 0)
        for j in range(2 * npair, nblk):
            c0 = issue(j * sb, buf_a0, buf_b0, sem_a0, sem_b0)
            drain(j * sb, *c0, buf_a0, buf_b0)
        if btail:
            c0 = issue(ch - sb, buf_a0, buf_b0, sem_a0, sem_b0)
            drain(ch - sb, *c0, buf_a0, buf_b0)

    return k(A, B, row, col, px, py, pz)


_DN_T = (((0,), (0,)), ((), ()))


def _emlp_body(ab_ref, rad_ref, eat_ref, wr_ref, wat_ref, b1_ref,
               we2_ref, b2_ref, m2_ref):
    f32 = jnp.float32
    m1 = (_unpack_sum(ab_ref[...])
          + jax.lax.dot_general(rad_ref[...], wr_ref[...], _DN_T,
                                preferred_element_type=f32)
          + jax.lax.dot_general(eat_ref[...], wat_ref[...], _DN_T,
                                preferred_element_type=f32)
          + b1_ref[...])
    em = _ssp(m1).astype(jnp.bfloat16)
    m2_ref[...] = _ssp(jnp.dot(em, we2_ref[...], preferred_element_type=f32)
                       + b2_ref[...])


def _emlp(ab, rad, eat, w_r, W_at, b_e1, W_e2, b_e2, be):
    e, h = ab.shape
    enf = eat.shape[0]
    grid = e // be
    full = lambda s: pl.BlockSpec(s, lambda i: (0, 0))
    return pl.pallas_call(
        _emlp_body,
        grid=(grid,),
        in_specs=[
            pl.BlockSpec((be, h), lambda i: (i, 0)),
            pl.BlockSpec((1, be), lambda i: (0, i)),
            pl.BlockSpec((enf, be), lambda i: (0, i)),
            full((1, h)), full((enf, h)), full((1, h)),
            full((h, h)), full((1, h)),
        ],
        out_specs=pl.BlockSpec((be, h), lambda i: (i, 0)),
        out_shape=jax.ShapeDtypeStruct((e, h), jnp.float32),
    )(ab, rad, eat, w_r.reshape(1, h), W_at,
      b_e1.reshape(1, h), W_e2.astype(jnp.bfloat16), b_e2.reshape(1, h))


def _sc_scatter(m2s, row3ds, n):
    nseg = len(m2s)
    e, h = m2s[0].shape
    ch = e // _NW
    nblk, sb = row3ds[0].shape[1], row3ds[0].shape[2]
    zb = 200
    nb_n = n // zb
    nrounds = -(-nb_n // _NS)
    mesh = plsc.VectorSubcoreMesh(core_axis_name="c", subcore_axis_name="s")

    @functools.partial(
        pl.kernel,
        out_type=jax.ShapeDtypeStruct((_NC, n, h), jnp.float32),
        mesh=mesh,
        scratch_types=[
            pltpu.VMEM((nblk, sb), jnp.int32),
            pltpu.VMEM((sb, h), jnp.float32),
            pltpu.VMEM((sb, h), jnp.float32),
            pltpu.VMEM((zb, h), jnp.float32),
            pltpu.VMEM_SHARED((n, h), jnp.float32),
            pltpu.SemaphoreType.DMA,
            pltpu.SemaphoreType.DMA,
        ],
    )
    def k(*refs):
        m2_hbms = refs[:nseg]
        row3d_hbms = refs[nseg:2 * nseg]
        agg_hbm = refs[2 * nseg]
        idx_v, buf_m0, buf_m1, zbuf, agg_sh, sem_m0, sem_m1 = \
            refs[2 * nseg + 1:]
        cid = lax.axis_index("c")
        sid = lax.axis_index("s")
        wid = sid * _NC + cid
        base = wid * ch

        zeros16 = jnp.zeros((16,), jnp.float32)

        def z_body(i, carry):
            for j in range(h // 16):
                zbuf[i, pl.ds(j * 16, 16)] = zeros16
            return carry

        lax.fori_loop(0, zb, z_body, 0)
        for t in range(nrounds):
            blk = sid + t * _NS
            if nb_n % _NS == 0:
                pltpu.sync_copy(zbuf, agg_sh.at[pl.ds(blk * zb, zb)])
            else:
                @pl.when(blk < nb_n)
                def _():
                    pltpu.sync_copy(zbuf, agg_sh.at[pl.ds(blk * zb, zb)])
        plsc.subcore_barrier()

        for m2_hbm, row3d_hbm in zip(m2_hbms, row3d_hbms):
            pltpu.sync_copy(row3d_hbm.at[wid], idx_v)

            def s_body(i, carry):
                j0, j1 = 2 * i, 2 * i + 1
                c0 = pltpu.async_copy(
                    m2_hbm.at[pl.ds(base + j0 * sb, sb)], buf_m0, sem_m0)
                c1 = pltpu.async_copy(
                    m2_hbm.at[pl.ds(base + j1 * sb, sb)], buf_m1, sem_m1)
                c0.wait()
                pltpu.sync_copy(buf_m0, agg_sh.at[idx_v.at[j0]], add=True)
                c1.wait()
                pltpu.sync_copy(buf_m1, agg_sh.at[idx_v.at[j1]], add=True)
                return carry

            lax.fori_loop(0, nblk // 2, s_body, 0)
            for j in range(2 * (nblk // 2), nblk):
                pltpu.sync_copy(m2_hbm.at[pl.ds(base + j * sb, sb)], buf_m0)
                pltpu.sync_copy(buf_m0, agg_sh.at[idx_v.at[j]], add=True)
        plsc.subcore_barrier()
        for t in range(nrounds):
            blk = sid + t * _NS
            if nb_n % _NS == 0:
                pltpu.sync_copy(agg_sh.at[pl.ds(blk * zb, zb)],
                                agg_hbm.at[cid, pl.ds(blk * zb, zb)])
            else:
                @pl.when(blk < nb_n)
                def _():
                    pltpu.sync_copy(agg_sh.at[pl.ds(blk * zb, zb)],
                                    agg_hbm.at[cid, pl.ds(blk * zb, zb)])

    return k(*m2s, *row3ds)


def _node_body(h0_ref, h_ref, *rest):
    f32 = jnp.float32
    agg_refs = rest[:-8]
    (wnh_ref, wna_ref, b1_ref, wn2_ref, b2_ref, wo_ref, bo_ref,
     out_ref) = rest[-8:]
    agg = agg_refs[0][0] + agg_refs[0][1]
    for r in agg_refs[1:]:
        agg = agg + r[0] + r[1]
    t = _ssp(jnp.dot(h_ref[...], wnh_ref[...], preferred_element_type=f32)
             + jnp.dot(agg, wna_ref[...], preferred_element_type=f32)
             + b1_ref[...])
    hn = jnp.dot(t, wn2_ref[...], preferred_element_type=f32) + b2_ref[...]
    hn = jnp.dot(hn, wo_ref[...], preferred_element_type=f32) + bo_ref[...]
    h0 = h0_ref[...]
    out_ref[0] = h0
    out_ref[1] = h0 + hn


def _node(h0, hfeat, aggps, Wn1h, Wn1a, b_n1, W_n2, b_n2, W_out, b_out, bn):
    n, h = h0.shape
    grid = n // bn
    full = lambda s: pl.BlockSpec(s, lambda i: (0, 0))
    return pl.pallas_call(
        _node_body,
        grid=(grid,),
        in_specs=[
            pl.BlockSpec((bn, h), lambda i: (i, 0)),
            pl.BlockSpec((bn, h), lambda i: (i, 0)),
        ] + [pl.BlockSpec((_NC, bn, h), lambda i: (0, i, 0))] * len(aggps) + [
            full((h, h)), full((h, h)), full((1, h)),
            full((h, h)), full((1, h)), full((h, h)), full((1, h)),
        ],
        out_specs=pl.BlockSpec((2, bn, h), lambda i: (0, i, 0)),
        out_shape=jax.ShapeDtypeStruct((2, n, h), jnp.float32),
    )(h0, hfeat, *aggps, Wn1h, Wn1a, b_n1.reshape(1, h), W_n2,
      b_n2.reshape(1, h), W_out, b_out.reshape(1, h))


def kernel(x, edge_index, pos, edge_attr,
           W_emb, b_emb, W_in, b_in,
           W_e1, b_e1, W_e2, b_e2,
           W_n1, b_n1, W_n2, b_n2,
           W_c1, b_c1, W_c2,
           W_out, b_out):
    n, d = x.shape
    h = W_emb.shape[1]
    e = edge_index.shape[1]
    unit = e // 250
    segs = [100 * unit, 100 * unit, 50 * unit]
    sbg = 80
    sbs = 40

    row = edge_index[0].astype(jnp.int32)
    col = edge_index[1].astype(jnp.int32)
    px = pos[:, 0].astype(jnp.float32)
    py = pos[:, 1].astype(jnp.float32)
    pz = pos[:, 2].astype(jnp.float32)

    We1a = W_e1[:h]
    We1b = W_e1[h:2 * h]
    w_r = W_e1[2 * h]
    W_at = W_e1[2 * h + 1:]

    h0, hfeat, A, B = _prep(x, W_emb, b_emb, W_in, b_in, We1a, We1b, bn=1000)
    eaT = edge_attr.T
    aggps = []
    off = 0
    for eseg in segs:
        sl = slice(off, off + eseg)
        off += eseg
        row_s, col_s = row[sl], col[sl]
        ab, rad = _sc_gather(A, B, row_s, col_s, px, py, pz, sbg)
        m2 = _emlp(ab, rad, eaT[:, sl], w_r, W_at, b_e1, W_e2, b_e2,
                   be=3200)
        row3d = row_s.reshape(_NW, eseg // (_NW * sbs), sbs)
        aggps.append(_sc_scatter([m2], [row3d], n))
    return _node(h0, hfeat, aggps, W_n1[:h], W_n1[h:], b_n1, W_n2, b_n2,
                 W_out, b_out, bn=1000)
